# baseline (speedup 1.0000x reference)
import numpy as np

import jax
import jax.numpy as jnp
from jax import lax
from jax.experimental import pallas as pl
from jax.experimental.pallas import tpu as pltpu

_B, _D, _H, _W = 4, 64, 256, 256
_SD, _SH, _SW = 16, 64, 64
_NHOLES = 4
_DBLK = 4


def _hole_offsets_np():
    base_key = jax.random.key(42)
    rows = []
    for hole_idx in range(1, _NHOLES + 1):
        k = jax.random.fold_in(base_key, hole_idx)
        kz, ky, kx = jax.random.split(k, 3)
        rows.append(np.stack([
            np.asarray(jax.random.randint(kz, (_B,), 0, _D - _SD + 1)),
            np.asarray(jax.random.randint(ky, (_B,), 0, _H - _SH + 1)),
            np.asarray(jax.random.randint(kx, (_B,), 0, _W - _SW + 1)),
        ]))
    return np.stack(rows).astype(np.int32)


def _hole_offsets_concrete():
    try:
        cpu = jax.local_devices(backend="cpu")[0]
        with jax.default_device(cpu):
            return _hole_offsets_np()
    except RuntimeError:
        return _hole_offsets_np()


_OFFS = _hole_offsets_concrete()


def _block_mask(offs_ref):
    b = pl.program_id(0)
    z0 = pl.program_id(1) * _DBLK
    ziota = lax.broadcasted_iota(jnp.int32, (1, _DBLK, 1, 1), 1) + z0
    yiota = lax.broadcasted_iota(jnp.int32, (1, 1, _H, _W), 2)
    xiota = lax.broadcasted_iota(jnp.int32, (1, 1, _H, _W), 3)
    mask = None
    for h in range(_NHOLES):
        bz = offs_ref[h, 0, b]
        by = offs_ref[h, 1, b]
        bx = offs_ref[h, 2, b]
        zm = (ziota >= bz) & (ziota < bz + _SD)
        ym = (yiota >= by) & (yiota < by + _SH)
        xm = (xiota >= bx) & (xiota < bx + _SW)
        m = zm & (ym & xm)
        mask = m if mask is None else mask | m
    return mask


def _all_kernel(offs_ref, vol_ref, gm_ref, gs_ref,
                vol_out, gm_out, gs_out, msk_out):
    mask = _block_mask(offs_ref)
    vol_out[...] = jnp.where(mask, jnp.float32(0.0), vol_ref[...])
    gm_out[...] = jnp.where(mask, jnp.float32(2.0), gm_ref[...])
    gs_out[...] = jnp.where(mask, jnp.float32(2.0), gs_ref[...])
    msk_out[...] = mask.astype(jnp.int8)


def _gs_msk_kernel(offs_ref, gs_ref, gs_out, msk_out):
    mask = _block_mask(offs_ref)
    gs_out[...] = jnp.where(mask, jnp.float32(2.0), gs_ref[...])
    msk_out[...] = mask.astype(jnp.int8)


def _call(body, n_in, n_out, inputs, out_shapes):
    offs = jnp.asarray(_OFFS)
    grid = (_B, _D // _DBLK)
    blk = (1, _DBLK, _H, _W)
    data_spec = pl.BlockSpec(blk, lambda b, d, offs: (b, d, 0, 0))
    grid_spec = pltpu.PrefetchScalarGridSpec(
        num_scalar_prefetch=1,
        grid=grid,
        in_specs=[data_spec] * n_in,
        out_specs=[data_spec] * n_out,
    )
    return pl.pallas_call(
        body,
        grid_spec=grid_spec,
        out_shape=out_shapes,
        compiler_params=pltpu.CompilerParams(
            vmem_limit_bytes=110 * 1024 * 1024,
        ),
    )(offs, *inputs)


@jax.jit
def kernel(volume, gt_mask, gt_skel):
    f32s = jax.ShapeDtypeStruct(volume.shape, jnp.float32)
    i8s = jax.ShapeDtypeStruct(volume.shape, jnp.int8)
    vol, gm, gs, msk = _call(_all_kernel, 3, 4, (volume, gt_mask, gt_skel),
                             (f32s, f32s, f32s, i8s))
    return vol, gm, gs, msk.astype(jnp.bool_)


# speedup vs baseline: 1.1478x; 1.1478x over previous
import numpy as np

import jax
import jax.numpy as jnp
from jax import lax
from jax.experimental import pallas as pl
from jax.experimental.pallas import tpu as pltpu

_B, _D, _H, _W = 4, 64, 256, 256
_SD, _SH, _SW = 16, 64, 64
_NHOLES = 4
_DBLK = 16


def _hole_offsets_np():
    base_key = jax.random.key(42)
    rows = []
    for hole_idx in range(1, _NHOLES + 1):
        k = jax.random.fold_in(base_key, hole_idx)
        kz, ky, kx = jax.random.split(k, 3)
        rows.append(np.stack([
            np.asarray(jax.random.randint(kz, (_B,), 0, _D - _SD + 1)),
            np.asarray(jax.random.randint(ky, (_B,), 0, _H - _SH + 1)),
            np.asarray(jax.random.randint(kx, (_B,), 0, _W - _SW + 1)),
        ]))
    return np.stack(rows).astype(np.int32)


def _hole_offsets_concrete():
    try:
        cpu = jax.local_devices(backend="cpu")[0]
        with jax.default_device(cpu):
            return _hole_offsets_np()
    except RuntimeError:
        return _hole_offsets_np()


_OFFS = _hole_offsets_concrete()


def _block_mask(offs_ref):
    b = pl.program_id(0)
    z0 = pl.program_id(1) * _DBLK
    ziota = lax.broadcasted_iota(jnp.int32, (1, _DBLK, 1, 1), 1) + z0
    yiota = lax.broadcasted_iota(jnp.int32, (1, 1, _H, _W), 2)
    xiota = lax.broadcasted_iota(jnp.int32, (1, 1, _H, _W), 3)
    mask = None
    for h in range(_NHOLES):
        bz = offs_ref[h, 0, b]
        by = offs_ref[h, 1, b]
        bx = offs_ref[h, 2, b]
        zm = (ziota >= bz) & (ziota < bz + _SD)
        ym = (yiota >= by) & (yiota < by + _SH)
        xm = (xiota >= bx) & (xiota < bx + _SW)
        m = zm & (ym & xm)
        mask = m if mask is None else mask | m
    return mask


def _all_kernel(offs_ref, vol_ref, gm_ref, gs_ref,
                vol_out, gm_out, gs_out, msk_out):
    mask = _block_mask(offs_ref)
    vol_out[...] = jnp.where(mask, jnp.float32(0.0), vol_ref[...])
    gm_out[...] = jnp.where(mask, jnp.float32(2.0), gm_ref[...])
    gs_out[...] = jnp.where(mask, jnp.float32(2.0), gs_ref[...])
    msk_out[...] = mask.astype(jnp.int8)


def _gs_msk_kernel(offs_ref, gs_ref, gs_out, msk_out):
    mask = _block_mask(offs_ref)
    gs_out[...] = jnp.where(mask, jnp.float32(2.0), gs_ref[...])
    msk_out[...] = mask.astype(jnp.int8)


def _call(body, n_in, n_out, inputs, out_shapes):
    offs = jnp.asarray(_OFFS)
    grid = (_B, _D // _DBLK)
    blk = (1, _DBLK, _H, _W)
    data_spec = pl.BlockSpec(blk, lambda b, d, offs: (b, d, 0, 0))
    grid_spec = pltpu.PrefetchScalarGridSpec(
        num_scalar_prefetch=1,
        grid=grid,
        in_specs=[data_spec] * n_in,
        out_specs=[data_spec] * n_out,
    )
    return pl.pallas_call(
        body,
        grid_spec=grid_spec,
        out_shape=out_shapes,
        compiler_params=pltpu.CompilerParams(
            vmem_limit_bytes=110 * 1024 * 1024,
        ),
    )(offs, *inputs)


@jax.jit
def kernel(volume, gt_mask, gt_skel):
    f32s = jax.ShapeDtypeStruct(volume.shape, jnp.float32)
    i8s = jax.ShapeDtypeStruct(volume.shape, jnp.int8)
    vol, gm, gs, msk = _call(_all_kernel, 3, 4, (volume, gt_mask, gt_skel),
                             (f32s, f32s, f32s, i8s))
    return vol, gm, gs, msk.astype(jnp.bool_)


# speedup vs baseline: 1.1487x; 1.0008x over previous
import numpy as np

import jax
import jax.numpy as jnp
from jax import lax
from jax.experimental import pallas as pl
from jax.experimental.pallas import tpu as pltpu

_B, _D, _H, _W = 4, 64, 256, 256
_SD, _SH, _SW = 16, 64, 64
_NHOLES = 4
_DBLK = 16


def _hole_offsets_np():
    base_key = jax.random.key(42)
    rows = []
    for hole_idx in range(1, _NHOLES + 1):
        k = jax.random.fold_in(base_key, hole_idx)
        kz, ky, kx = jax.random.split(k, 3)
        rows.append(np.stack([
            np.asarray(jax.random.randint(kz, (_B,), 0, _D - _SD + 1)),
            np.asarray(jax.random.randint(ky, (_B,), 0, _H - _SH + 1)),
            np.asarray(jax.random.randint(kx, (_B,), 0, _W - _SW + 1)),
        ]))
    return np.stack(rows).astype(np.int32)


def _hole_offsets_concrete():
    try:
        cpu = jax.local_devices(backend="cpu")[0]
        with jax.default_device(cpu):
            return _hole_offsets_np()
    except RuntimeError:
        return _hole_offsets_np()


_OFFS = _hole_offsets_concrete()


def _block_mask(offs_ref):
    b = pl.program_id(0)
    z0 = pl.program_id(1) * _DBLK
    ziota = lax.broadcasted_iota(jnp.int32, (1, _DBLK, 1, 1), 1) + z0
    yiota = lax.broadcasted_iota(jnp.int32, (1, 1, _H, _W), 2)
    xiota = lax.broadcasted_iota(jnp.int32, (1, 1, _H, _W), 3)
    mask = None
    for h in range(_NHOLES):
        bz = offs_ref[h, 0, b]
        by = offs_ref[h, 1, b]
        bx = offs_ref[h, 2, b]
        zm = (ziota >= bz) & (ziota < bz + _SD)
        ym = (yiota >= by) & (yiota < by + _SH)
        xm = (xiota >= bx) & (xiota < bx + _SW)
        m = zm & (ym & xm)
        mask = m if mask is None else mask | m
    return mask


def _all_kernel(offs_ref, vol_ref, gm_ref, gs_ref,
                vol_out, gm_out, gs_out, msk_out):
    mask = _block_mask(offs_ref)
    vol_out[...] = jnp.where(mask, jnp.float32(0.0), vol_ref[...])
    gm_out[...] = jnp.where(mask, jnp.float32(2.0), gm_ref[...])
    gs_out[...] = jnp.where(mask, jnp.float32(2.0), gs_ref[...])
    msk_out[...] = mask.astype(jnp.int8)


def _call(body, n_in, n_out, inputs, out_shapes):
    offs = jnp.asarray(_OFFS)
    grid = (_B, _D // _DBLK)
    blk = (1, _DBLK, _H, _W)
    data_spec = pl.BlockSpec(blk, lambda b, d, offs: (b, d, 0, 0))
    grid_spec = pltpu.PrefetchScalarGridSpec(
        num_scalar_prefetch=1,
        grid=grid,
        in_specs=[data_spec] * n_in,
        out_specs=[data_spec] * n_out,
    )
    return pl.pallas_call(
        body,
        grid_spec=grid_spec,
        out_shape=out_shapes,
        compiler_params=pltpu.CompilerParams(
            vmem_limit_bytes=110 * 1024 * 1024,
        ),
    )(offs, *inputs)


@jax.jit
def kernel(volume, gt_mask, gt_skel):
    f32s = jax.ShapeDtypeStruct(volume.shape, jnp.float32)
    i8s = jax.ShapeDtypeStruct(volume.shape, jnp.int8)
    vol, gm, gs, msk = _call(_all_kernel, 3, 4, (volume, gt_mask, gt_skel),
                             (f32s, f32s, f32s, i8s))
    return vol, gm, gs, msk.astype(jnp.bool_)
